# baseline (speedup 1.0000x reference)
import functools

import jax
import jax.numpy as jnp
from jax import lax
from jax.experimental import pallas as pl
from jax.experimental.pallas import tpu as pltpu
from jax.experimental.pallas import tpu_sc as plsc

B = 16384
F = 32
NW = 32
BPW = B // NW
CHUNK = 128
NCHUNK = BPW // CHUNK
L = 16


def _make_sc_kernel():
    mesh = plsc.VectorSubcoreMesh(core_axis_name="c", subcore_axis_name="s")

    @functools.partial(
        pl.kernel,
        mesh=mesh,
        out_type=jax.ShapeDtypeStruct((B,), jnp.float32),
        compiler_params=pltpu.CompilerParams(
            needs_layout_passes=False, use_tc_tiling_on_sc=False),
        scratch_types=[
            pltpu.VMEM((NCHUNK, CHUNK), jnp.int32),
            pltpu.VMEM((NCHUNK, CHUNK), jnp.int32),
            pltpu.VMEM((BPW, F), jnp.float32),
            pltpu.VMEM((BPW, F), jnp.float32),
            pltpu.VMEM((BPW,), jnp.float32),
            pltpu.VMEM((3 * L,), jnp.float32),
            pltpu.SemaphoreType.DMA,
        ],
    )
    def sc_kernel(user_hbm, item_hbm, uemb_hbm, iemb_hbm, params_hbm,
                  out_hbm, uidx, iidx, urows, irows, outv, pv, sem):
        wid = lax.axis_index("s") * 2 + lax.axis_index("c")

        pltpu.sync_copy(user_hbm.at[pl.ds(wid * NCHUNK, NCHUNK)], uidx)
        pltpu.sync_copy(item_hbm.at[pl.ds(wid * NCHUNK, NCHUNK)], iidx)
        pltpu.sync_copy(params_hbm, pv)

        copies = []
        for c in range(NCHUNK):
            copies.append(pltpu.async_copy(
                uemb_hbm.at[uidx.at[c]], urows.at[pl.ds(c * CHUNK, CHUNK)], sem))
            copies.append(pltpu.async_copy(
                iemb_hbm.at[iidx.at[c]], irows.at[pl.ds(c * CHUNK, CHUNK)], sem))
        for cp in copies:
            cp.wait()

        w0 = pv[pl.ds(0, L)]
        w1 = pv[pl.ds(L, L)]
        bv = pv[pl.ds(2 * L, L)]
        iota = lax.iota(jnp.int32, L)

        def group(g, carry):
            rows = g * L + iota
            mf = jnp.zeros((L,), jnp.float32)
            d2 = jnp.zeros((L,), jnp.float32)
            for f in range(F):
                colf = jnp.full((L,), f, jnp.int32)
                u = plsc.load_gather(urows, [rows, colf])
                i = plsc.load_gather(irows, [rows, colf])
                mf = mf + u * i
                d = u - i
                d2 = d2 + d * d
            bits = lax.bitcast_convert_type(d2, jnp.int32)
            r = lax.bitcast_convert_type(
                jnp.int32(0x5F3759DF) - (bits >> 1), jnp.float32)
            for _ in range(3):
                r = r * (1.5 - 0.5 * d2 * r * r)
            p2 = d2 * r
            outv[pl.ds(g * L, L)] = w0 * p2 + w1 * mf + bv
            return carry

        lax.fori_loop(0, BPW // L, group, 0)
        pltpu.sync_copy(outv, out_hbm.at[pl.ds(wid * BPW, BPW)])

    return sc_kernel


_SC_KERNEL = _make_sc_kernel()


def kernel(user, item, user_emb, item_emb, W, b):
    user2d = user.reshape(NW * NCHUNK, CHUNK)
    item2d = item.reshape(NW * NCHUNK, CHUNK)
    params = jnp.concatenate([
        jnp.full((L,), W[0, 0], jnp.float32),
        jnp.full((L,), W[0, 1], jnp.float32),
        jnp.full((L,), b[0], jnp.float32),
    ])
    return _SC_KERNEL(user2d, item2d, user_emb, item_emb, params)
